# baseline (speedup 1.0000x reference)
import functools

import jax
import jax.numpy as jnp
from jax import lax
from jax.experimental import pallas as pl
from jax.experimental.pallas import tpu as pltpu
from jax.experimental.pallas import tpu_sc as plsc

B = 16384
D = 64

_NC = 2
_NS = 16
_NW = _NC * _NS
_BPW = B // _NW
_CHUNK = 128
_NCHUNK = _BPW // _CHUNK


def _sc_gather_body(uidx_hbm, iidx_hbm, wt_hbm, ht_hbm, u_out, v_out,
                    uidx_v, iidx_v, urows, vrows, su, sv):
    wid = lax.axis_index("s") * _NC + lax.axis_index("c")
    base = wid * _BPW
    pltpu.sync_copy(uidx_hbm.at[pl.ds(wid * _NCHUNK, _NCHUNK)], uidx_v)
    pltpu.sync_copy(iidx_hbm.at[pl.ds(wid * _NCHUNK, _NCHUNK)], iidx_v)
    copies = []
    for j in range(_NCHUNK):
        copies.append(pltpu.async_copy(
            wt_hbm.at[uidx_v.at[j]], urows.at[pl.ds(j * _CHUNK, _CHUNK)], su))
        copies.append(pltpu.async_copy(
            ht_hbm.at[iidx_v.at[j]], vrows.at[pl.ds(j * _CHUNK, _CHUNK)], sv))
    for c in copies:
        c.wait()
    pltpu.sync_copy(urows, u_out.at[pl.ds(base, _BPW)])
    pltpu.sync_copy(vrows, v_out.at[pl.ds(base, _BPW)])


@functools.lru_cache(maxsize=1)
def _sc_gather():
    return pl.kernel(
        _sc_gather_body,
        out_type=(jax.ShapeDtypeStruct((B, D), jnp.float32),
                  jax.ShapeDtypeStruct((B, D), jnp.float32)),
        mesh=plsc.VectorSubcoreMesh(core_axis_name="c", subcore_axis_name="s"),
        compiler_params=pltpu.CompilerParams(use_tc_tiling_on_sc=False),
        scratch_types=[
            pltpu.VMEM((_NCHUNK, _CHUNK), jnp.int32),
            pltpu.VMEM((_NCHUNK, _CHUNK), jnp.int32),
            pltpu.VMEM((_BPW, D), jnp.float32),
            pltpu.VMEM((_BPW, D), jnp.float32),
            pltpu.SemaphoreType.DMA,
            pltpu.SemaphoreType.DMA,
        ],
    )


_BLK = 2048


def _mlp_body(u_ref, v_ref, w1u_ref, w1v_ref, b_ref, w2_ref, o_ref):
    h = (jnp.dot(u_ref[...], w1u_ref[...],
                 preferred_element_type=jnp.float32,
                 precision=lax.Precision.HIGHEST)
         + jnp.dot(v_ref[...], w1v_ref[...],
                   preferred_element_type=jnp.float32,
                   precision=lax.Precision.HIGHEST)
         + b_ref[...])
    h = jnp.maximum(h, 0.0)
    o_ref[...] = jnp.dot(h, w2_ref[...],
                         preferred_element_type=jnp.float32,
                         precision=lax.Precision.HIGHEST)


_mlp = pl.pallas_call(
    _mlp_body,
    grid=(B // _BLK,),
    in_specs=[
        pl.BlockSpec((_BLK, D), lambda i: (i, 0)),
        pl.BlockSpec((_BLK, D), lambda i: (i, 0)),
        pl.BlockSpec((D, D), lambda i: (0, 0)),
        pl.BlockSpec((D, D), lambda i: (0, 0)),
        pl.BlockSpec((1, D), lambda i: (0, 0)),
        pl.BlockSpec((D, 1), lambda i: (0, 0)),
    ],
    out_specs=pl.BlockSpec((_BLK, 1), lambda i: (i, 0)),
    out_shape=jax.ShapeDtypeStruct((B, 1), jnp.float32),
)


def kernel(x, W_table, H_table, lin1_w, lin1_b, lin2_w):
    uidx = x[:, 0].reshape(B // _CHUNK, _CHUNK)
    iidx = x[:, 1].reshape(B // _CHUNK, _CHUNK)
    u_emb, v_emb = _sc_gather()(uidx, iidx, W_table, H_table)
    w1t = lin1_w.T
    return _mlp(u_emb, v_emb, w1t[:D], w1t[D:], lin1_b.reshape(1, D),
                lin2_w.T)


# speedup vs baseline: 1.5614x; 1.5614x over previous
import functools

import jax
import jax.numpy as jnp
from jax import lax
from jax.experimental import pallas as pl
from jax.experimental.pallas import tpu as pltpu
from jax.experimental.pallas import tpu_sc as plsc

B = 16384
D = 64

_NC = 2
_NS = 16
_NW = _NC * _NS
_BPW = B // _NW
_NGRP = _BPW // 16


def _sc_gather_body(uidx_hbm, iidx_hbm, wt_hbm, ht_hbm, u_out, v_out,
                    idx_v, rows_v, sem):
    wid = lax.axis_index("s") * _NC + lax.axis_index("c")
    base = wid * _BPW
    lane = lax.iota(jnp.int32, 16)

    def do_table(idx_hbm, tab_hbm, out_hbm):
        pltpu.sync_copy(idx_hbm.at[pl.ds(base, _BPW)], idx_v)

        def group(g, carry):
            chunk = idx_v[pl.ds(g * 16, 16)]
            for j in range(16):
                s = chunk[j]
                pltpu.async_copy(tab_hbm.at[pl.ds(s, 1)],
                                 rows_v.at[pl.ds(g * 16 + j, 1)], sem)
            return carry

        lax.fori_loop(0, _NGRP, group, 0)
        pltpu.make_async_copy(tab_hbm.at[pl.ds(0, _BPW)], rows_v, sem).wait()
        pltpu.sync_copy(rows_v, out_hbm.at[pl.ds(base, _BPW)])

    do_table(uidx_hbm, wt_hbm, u_out)
    do_table(iidx_hbm, ht_hbm, v_out)


@functools.lru_cache(maxsize=1)
def _sc_gather():
    return pl.kernel(
        _sc_gather_body,
        out_type=(jax.ShapeDtypeStruct((B, D), jnp.float32),
                  jax.ShapeDtypeStruct((B, D), jnp.float32)),
        mesh=plsc.VectorSubcoreMesh(core_axis_name="c", subcore_axis_name="s"),
        scratch_types=[
            pltpu.VMEM((_BPW,), jnp.int32),
            pltpu.VMEM((_BPW, D), jnp.float32),
            pltpu.SemaphoreType.DMA,
        ],
    )


_BLK = 2048


def _mlp_body(u_ref, v_ref, w1u_ref, w1v_ref, b_ref, w2_ref, o_ref):
    h = (jnp.dot(u_ref[...], w1u_ref[...],
                 preferred_element_type=jnp.float32,
                 precision=lax.Precision.HIGHEST)
         + jnp.dot(v_ref[...], w1v_ref[...],
                   preferred_element_type=jnp.float32,
                   precision=lax.Precision.HIGHEST)
         + b_ref[...])
    h = jnp.maximum(h, 0.0)
    o_ref[...] = jnp.dot(h, w2_ref[...],
                         preferred_element_type=jnp.float32,
                         precision=lax.Precision.HIGHEST)


_mlp = pl.pallas_call(
    _mlp_body,
    grid=(B // _BLK,),
    in_specs=[
        pl.BlockSpec((_BLK, D), lambda i: (i, 0)),
        pl.BlockSpec((_BLK, D), lambda i: (i, 0)),
        pl.BlockSpec((D, D), lambda i: (0, 0)),
        pl.BlockSpec((D, D), lambda i: (0, 0)),
        pl.BlockSpec((1, D), lambda i: (0, 0)),
        pl.BlockSpec((D, 1), lambda i: (0, 0)),
    ],
    out_specs=pl.BlockSpec((_BLK, 1), lambda i: (i, 0)),
    out_shape=jax.ShapeDtypeStruct((B, 1), jnp.float32),
)


def kernel(x, W_table, H_table, lin1_w, lin1_b, lin2_w):
    uidx = x[:, 0]
    iidx = x[:, 1]
    u_emb, v_emb = _sc_gather()(uidx, iidx, W_table, H_table)
    w1t = lin1_w.T
    return _mlp(u_emb, v_emb, w1t[:D], w1t[D:], lin1_b.reshape(1, D),
                lin2_w.T)
